# Initial kernel scaffold; baseline (speedup 1.0000x reference)
import jax
import jax.numpy as jnp
from jax.experimental import pallas as pl


def kernel(input, table, W):
    raise NotImplementedError("write your pallas kernel here")


# speedup vs baseline: 1.1158x; 1.1158x over previous
import functools

import jax
import jax.numpy as jnp
from jax import lax
from jax.experimental import pallas as pl
from jax.experimental.pallas import tpu as pltpu
from jax.experimental.pallas import tpu_sc as plsc

_CW = 128


def _make_sc_gather(num_rows, emb_dim, num_workers, chunks_per_worker, table_dtype):
    rows_per_worker = chunks_per_worker * _CW
    mesh = plsc.VectorSubcoreMesh(core_axis_name="c", subcore_axis_name="s")
    num_cores = plsc.get_sparse_core_info().num_cores

    @functools.partial(
        pl.kernel,
        mesh=mesh,
        out_type=jax.ShapeDtypeStruct((num_rows, emb_dim), table_dtype),
        scratch_types=[
            pltpu.VMEM((chunks_per_worker, _CW), jnp.int32),
            pltpu.VMEM((_CW, emb_dim), table_dtype),
            pltpu.SemaphoreType.DMA,
        ],
        compiler_params=pltpu.CompilerParams(use_tc_tiling_on_sc=False),
    )
    def gather_k(table_hbm, idx_hbm, out_hbm, idx_v, rows_v, sem):
        wid = lax.axis_index("s") * num_cores + lax.axis_index("c")
        pltpu.sync_copy(
            idx_hbm.at[pl.ds(wid * chunks_per_worker, chunks_per_worker)], idx_v
        )
        base_row = wid * rows_per_worker

        def body(j, carry):
            pltpu.async_copy(table_hbm.at[idx_v.at[j]], rows_v, sem).wait()
            pltpu.sync_copy(rows_v, out_hbm.at[pl.ds(base_row + j * _CW, _CW)])
            return carry

        lax.fori_loop(0, chunks_per_worker, body, 0)

    return gather_k


def _mm_body(emb_ref, w_ref, out_ref):
    out_ref[...] = lax.dot_general(
        emb_ref[...], w_ref[...], (((1,), (1,)), ((), ())),
        preferred_element_type=jnp.float32,
    )


def kernel(input, table, W):
    B, L = input.shape
    V, D = table.shape
    O = W.shape[0]
    n = B * L

    idx = input.reshape(n // _CW, _CW).astype(jnp.int32)

    num_workers = 32
    chunks_per_worker = n // (_CW * num_workers)
    gather_k = _make_sc_gather(n, D, num_workers, chunks_per_worker, table.dtype)
    emb = gather_k(table, idx)

    BM = 8192
    out = pl.pallas_call(
        _mm_body,
        grid=(n // BM,),
        in_specs=[
            pl.BlockSpec((BM, D), lambda i: (i, 0)),
            pl.BlockSpec((O, D), lambda i: (0, 0)),
        ],
        out_specs=pl.BlockSpec((BM, O), lambda i: (i, 0)),
        out_shape=jax.ShapeDtypeStruct((n, O), jnp.float32),
    )(emb, W)

    return out.reshape(B, L, O)


# speedup vs baseline: 2.1935x; 1.9658x over previous
import functools

import jax
import jax.numpy as jnp
from jax import lax
from jax.experimental import pallas as pl
from jax.experimental.pallas import tpu as pltpu
from jax.experimental.pallas import tpu_sc as plsc

_CW = 128
_BN = 16384


def _make_sc_gather(num_rows, emb_dim, table_rows, chunks_per_worker, table_dtype):
    rows_per_worker = chunks_per_worker * _CW
    mesh = plsc.VectorSubcoreMesh(core_axis_name="c", subcore_axis_name="s")
    num_cores = plsc.get_sparse_core_info().num_cores

    @functools.partial(
        pl.kernel,
        mesh=mesh,
        out_type=jax.ShapeDtypeStruct((num_rows, emb_dim), table_dtype),
        scratch_types=[
            pltpu.VMEM((chunks_per_worker, _CW), jnp.int32),
            pltpu.VMEM((_CW, emb_dim), table_dtype),
            pltpu.SemaphoreType.DMA,
        ],
        compiler_params=pltpu.CompilerParams(use_tc_tiling_on_sc=False),
    )
    def gather_k(table_hbm, idx_hbm, out_hbm, idx_v, rows_v, sem):
        wid = lax.axis_index("s") * num_cores + lax.axis_index("c")
        pltpu.sync_copy(
            idx_hbm.at[pl.ds(wid * chunks_per_worker, chunks_per_worker)], idx_v
        )
        base_row = wid * rows_per_worker

        def body(j, carry):
            pltpu.async_copy(table_hbm.at[idx_v.at[j]], rows_v, sem).wait()
            pltpu.sync_copy(rows_v, out_hbm.at[pl.ds(base_row + j * _CW, _CW)])
            return carry

        lax.fori_loop(0, chunks_per_worker, body, 0)

    return gather_k


def _fuse_body(t_ref, w_ref, out_ref):
    dn = (((0,), (1,)), ((), ()))
    proj = lax.dot_general(
        t_ref[...], w_ref[...], dn, preferred_element_type=jnp.float32
    )
    out_ref[...] = jnp.concatenate(
        [proj[: _BN // 2], proj[_BN // 2 :]], axis=1
    )


def kernel(input, table, W):
    B, L = input.shape
    V, D = table.shape
    O = W.shape[0]
    n = B * L

    nblk = (V + _BN - 1) // _BN
    vpad = nblk * _BN

    idx = input.reshape(-1).astype(jnp.int32)
    j = idx % _BN
    ridx = (idx - j) + 2 * (j % (_BN // 2)) + j // (_BN // 2)
    ridx = ridx.reshape(n // _CW, _CW)

    tableT = table.T
    fused2 = pl.pallas_call(
        _fuse_body,
        grid=(nblk,),
        in_specs=[
            pl.BlockSpec((D, _BN), lambda i: (0, i)),
            pl.BlockSpec((O, D), lambda i: (0, 0)),
        ],
        out_specs=pl.BlockSpec((_BN // 2, 2 * O), lambda i: (i, 0)),
        out_shape=jax.ShapeDtypeStruct((vpad // 2, 2 * O), jnp.float32),
    )(tableT, W)
    fused_lin = fused2.reshape(vpad, O)

    chunks_per_worker = n // (_CW * 32)
    gather_k = _make_sc_gather(n, O, vpad, chunks_per_worker, fused_lin.dtype)
    out = gather_k(fused_lin, ridx)

    return out.reshape(B, L, O)
